# Initial kernel scaffold; baseline (speedup 1.0000x reference)
import jax
import jax.numpy as jnp
from jax.experimental import pallas as pl


def kernel(encoded_obs, actives, W_in, b_in, W_qkv, b_qkv, W_o, b_o, W_out, b_out, W_v, b_v):
    raise NotImplementedError("write your pallas kernel here")


# speedup vs baseline: 2.4523x; 2.4523x over previous
import math

import jax
import jax.numpy as jnp
from jax.experimental import pallas as pl
from jax.experimental.pallas import tpu as pltpu

_B, _N, _D, _E, _H = 64, 256, 256, 256, 8
_DH = _E // _H
_NEG = -1e9
_SCALE = 1.0 / math.sqrt(_DH)


def _critic_body(act_ref, obs_ref, wd_ref, wlast_ref, bin_ref, wqkv_ref,
                 bqkv_ref, wo_ref, bo_ref, wout_ref, bout_ref, wv_ref,
                 bv_ref, out_ref):
    b = pl.program_id(0)
    a = act_ref[b]
    ratio = a.astype(jnp.float32) * (1.0 / _N)

    obs = obs_ref[0]
    h = jnp.dot(obs, wd_ref[...], preferred_element_type=jnp.float32)
    h = h + (ratio * wlast_ref[...] + bin_ref[...])
    h = jnp.maximum(h, 0.0)

    qkv = jnp.dot(h, wqkv_ref[...], preferred_element_type=jnp.float32)
    qkv = qkv + bqkv_ref[...]

    lane = jax.lax.broadcasted_iota(jnp.int32, (1, _N), 1)
    kmask = lane < a
    addmask = jnp.where(kmask, 0.0, _NEG)

    parts = []
    for i in range(_H):
        qh = qkv[:, i * _DH:(i + 1) * _DH]
        kh = qkv[:, _E + i * _DH:_E + (i + 1) * _DH]
        vh = qkv[:, 2 * _E + i * _DH:2 * _E + (i + 1) * _DH]
        s = jax.lax.dot_general(qh, kh, (((1,), (1,)), ((), ())),
                                preferred_element_type=jnp.float32)
        s = s * _SCALE + addmask
        m = jnp.max(s, axis=1, keepdims=True)
        e = jnp.exp(s - m)
        p = e / jnp.sum(e, axis=1, keepdims=True)
        parts.append(jnp.dot(p, vh, preferred_element_type=jnp.float32))
    ctx = jnp.concatenate(parts, axis=1)

    attn_out = jnp.dot(ctx, wo_ref[...],
                       preferred_element_type=jnp.float32) + bo_ref[...]
    rsa = jnp.dot(attn_out + h, wout_ref[...],
                  preferred_element_type=jnp.float32)
    rsa = jnp.maximum(rsa + bout_ref[...], 0.0)

    vrow = jax.lax.dot_general(wv_ref[...], rsa, (((1,), (1,)), ((), ())),
                               preferred_element_type=jnp.float32)
    vrow = vrow + bv_ref[...]
    vrow = jnp.where(vrow >= 0, vrow, 0.01 * vrow)
    vrow = jnp.where(kmask, vrow, 0.0)
    out_ref[0] = jnp.sum(vrow, axis=1, keepdims=True)


def kernel(encoded_obs, actives, W_in, b_in, W_qkv, b_qkv, W_o, b_o,
           W_out, b_out, W_v, b_v):
    acts = actives.reshape(_B).astype(jnp.int32)
    grid_spec = pltpu.PrefetchScalarGridSpec(
        num_scalar_prefetch=1,
        grid=(_B,),
        in_specs=[
            pl.BlockSpec((1, _N, _D), lambda b, *_: (b, 0, 0)),
            pl.BlockSpec((_D, _E), lambda b, *_: (0, 0)),
            pl.BlockSpec((1, _E), lambda b, *_: (0, 0)),
            pl.BlockSpec((1, _E), lambda b, *_: (0, 0)),
            pl.BlockSpec((_E, 3 * _E), lambda b, *_: (0, 0)),
            pl.BlockSpec((1, 3 * _E), lambda b, *_: (0, 0)),
            pl.BlockSpec((_E, _E), lambda b, *_: (0, 0)),
            pl.BlockSpec((1, _E), lambda b, *_: (0, 0)),
            pl.BlockSpec((_E, _E), lambda b, *_: (0, 0)),
            pl.BlockSpec((1, _E), lambda b, *_: (0, 0)),
            pl.BlockSpec((1, _E), lambda b, *_: (0, 0)),
            pl.BlockSpec((1, 1), lambda b, *_: (0, 0)),
        ],
        out_specs=pl.BlockSpec((1, 1, 1), lambda b, *_: (b, 0, 0)),
    )
    out = pl.pallas_call(
        _critic_body,
        grid_spec=grid_spec,
        out_shape=jax.ShapeDtypeStruct((_B, 1, 1), jnp.float32),
        compiler_params=pltpu.CompilerParams(
            dimension_semantics=("parallel",)),
        name="critic_fused",
    )(acts, encoded_obs, W_in[:, :_D].T, W_in[:, _D].reshape(1, _E),
      b_in.reshape(1, _E), W_qkv.T, b_qkv.reshape(1, 3 * _E), W_o.T,
      b_o.reshape(1, _E), W_out.T, b_out.reshape(1, _E), W_v,
      b_v.reshape(1, 1))
    return out.reshape(_B, 1)


# speedup vs baseline: 3.0142x; 1.2291x over previous
import math

import jax
import jax.numpy as jnp
from jax.experimental import pallas as pl
from jax.experimental.pallas import tpu as pltpu

_B, _N, _D, _E, _H = 64, 256, 256, 256, 8
_DH = _E // _H
_NEG = -1e9
_SCALE = 1.0 / math.sqrt(_DH)
_G = 4


def _one_sample(a, obs, wd_ref, wlast_ref, bin_ref, wqkv_ref, bqkv_ref,
                wo_ref, bo_ref, wout_ref, bout_ref, wv_ref, bv_ref):
    ratio = a.astype(jnp.float32) * (1.0 / _N)
    h = jnp.dot(obs, wd_ref[...], preferred_element_type=jnp.float32)
    h = h + (ratio * wlast_ref[...] + bin_ref[...])
    h = jnp.maximum(h, 0.0)

    qkv = jnp.dot(h, wqkv_ref[...], preferred_element_type=jnp.float32)
    qkv = qkv + bqkv_ref[...]

    lane = jax.lax.broadcasted_iota(jnp.int32, (1, _N), 1)
    kmask = lane < a
    addmask = jnp.where(kmask, 0.0, _NEG)

    parts = []
    for i in range(_H):
        qh = qkv[:, i * _DH:(i + 1) * _DH]
        kh = qkv[:, _E + i * _DH:_E + (i + 1) * _DH]
        vh = qkv[:, 2 * _E + i * _DH:2 * _E + (i + 1) * _DH]
        s = jax.lax.dot_general(qh, kh, (((1,), (1,)), ((), ())),
                                preferred_element_type=jnp.float32)
        s = s * _SCALE + addmask
        m = jnp.max(s, axis=1, keepdims=True)
        e = jnp.exp(s - m)
        r = 1.0 / jnp.sum(e, axis=1, keepdims=True)
        ctx_h = jnp.dot(e, vh, preferred_element_type=jnp.float32)
        parts.append(ctx_h * r)
    ctx = jnp.concatenate(parts, axis=1)

    attn_out = jnp.dot(ctx, wo_ref[...],
                       preferred_element_type=jnp.float32) + bo_ref[...]
    rsa = jnp.dot(attn_out + h, wout_ref[...],
                  preferred_element_type=jnp.float32)
    rsa = jnp.maximum(rsa + bout_ref[...], 0.0)

    vrow = jax.lax.dot_general(wv_ref[...], rsa, (((1,), (1,)), ((), ())),
                               preferred_element_type=jnp.float32)
    vrow = vrow + bv_ref[...]
    vrow = jnp.where(vrow >= 0, vrow, 0.01 * vrow)
    vrow = jnp.where(kmask, vrow, 0.0)
    return jnp.sum(vrow, axis=1, keepdims=True)


def _critic_body(act_ref, obs_ref, wd_ref, wlast_ref, bin_ref, wqkv_ref,
                 bqkv_ref, wo_ref, bo_ref, wout_ref, bout_ref, wv_ref,
                 bv_ref, out_ref):
    b = pl.program_id(0)
    for g in range(_G):
        a = act_ref[b * _G + g]
        out_ref[g] = _one_sample(
            a, obs_ref[g], wd_ref, wlast_ref, bin_ref, wqkv_ref, bqkv_ref,
            wo_ref, bo_ref, wout_ref, bout_ref, wv_ref, bv_ref)


def kernel(encoded_obs, actives, W_in, b_in, W_qkv, b_qkv, W_o, b_o,
           W_out, b_out, W_v, b_v):
    acts = actives.reshape(_B).astype(jnp.int32)
    grid_spec = pltpu.PrefetchScalarGridSpec(
        num_scalar_prefetch=1,
        grid=(_B // _G,),
        in_specs=[
            pl.BlockSpec((_G, _N, _D), lambda b, *_: (b, 0, 0)),
            pl.BlockSpec((_D, _E), lambda b, *_: (0, 0)),
            pl.BlockSpec((1, _E), lambda b, *_: (0, 0)),
            pl.BlockSpec((1, _E), lambda b, *_: (0, 0)),
            pl.BlockSpec((_E, 3 * _E), lambda b, *_: (0, 0)),
            pl.BlockSpec((1, 3 * _E), lambda b, *_: (0, 0)),
            pl.BlockSpec((_E, _E), lambda b, *_: (0, 0)),
            pl.BlockSpec((1, _E), lambda b, *_: (0, 0)),
            pl.BlockSpec((_E, _E), lambda b, *_: (0, 0)),
            pl.BlockSpec((1, _E), lambda b, *_: (0, 0)),
            pl.BlockSpec((1, _E), lambda b, *_: (0, 0)),
            pl.BlockSpec((1, 1), lambda b, *_: (0, 0)),
        ],
        out_specs=pl.BlockSpec((_G, 1, 1), lambda b, *_: (b, 0, 0)),
    )
    out = pl.pallas_call(
        _critic_body,
        grid_spec=grid_spec,
        out_shape=jax.ShapeDtypeStruct((_B, 1, 1), jnp.float32),
        compiler_params=pltpu.CompilerParams(
            dimension_semantics=("parallel",)),
        name="critic_fused",
    )(acts, encoded_obs, W_in[:, :_D].T, W_in[:, _D].reshape(1, _E),
      b_in.reshape(1, _E), W_qkv.T, b_qkv.reshape(1, 3 * _E), W_o.T,
      b_o.reshape(1, _E), W_out.T, b_out.reshape(1, _E), W_v,
      b_v.reshape(1, 1))
    return out.reshape(_B, 1)
